# baseline (speedup 1.0000x reference)
import functools

import jax
import jax.numpy as jnp
from jax.experimental import pallas as pl
from jax.experimental.pallas import tpu as pltpu


B, D, H, R = 16384, 2048, 128, 16
TB = 512


def _router_kernel(x_ref, w1_ref, b1_ref, w2_ref, b2_ref, rb_ref,
                   sel_ref, prob_ref):
    x = x_ref[...]
    h = jnp.maximum(
        jnp.dot(x, w1_ref[...], preferred_element_type=jnp.float32)
        + b1_ref[...], 0.0)
    logits = (jnp.dot(h, w2_ref[...], preferred_element_type=jnp.float32)
              + b2_ref[...] + rb_ref[...])
    m = jnp.max(logits, axis=-1, keepdims=True)
    e = jnp.exp(logits - m)
    prob_ref[...] = e * (1.0 / jnp.sum(e, axis=-1, keepdims=True))
    lane = jax.lax.broadcasted_iota(jnp.int32, logits.shape, 1)
    sel_ref[...] = jnp.min(jnp.where(logits == m, lane, R), axis=-1)


@functools.partial(jax.jit, static_argnames=())
def kernel(x, W1, b1, W2, b2, route_bias):
    grid = (B // TB,)
    sel, probs = pl.pallas_call(
        _router_kernel,
        grid=grid,
        in_specs=[
            pl.BlockSpec((TB, D), lambda i: (i, 0)),
            pl.BlockSpec((D, H), lambda i: (0, 0)),
            pl.BlockSpec((1, H), lambda i: (0, 0)),
            pl.BlockSpec((H, R), lambda i: (0, 0)),
            pl.BlockSpec((1, R), lambda i: (0, 0)),
            pl.BlockSpec((1, R), lambda i: (0, 0)),
        ],
        out_specs=[
            pl.BlockSpec((TB,), lambda i: (i,)),
            pl.BlockSpec((TB, R), lambda i: (i, 0)),
        ],
        out_shape=[
            jax.ShapeDtypeStruct((B,), jnp.int32),
            jax.ShapeDtypeStruct((B, R), jnp.float32),
        ],
        compiler_params=pltpu.CompilerParams(
            dimension_semantics=("parallel",)),
    )(x, W1, b1.reshape(1, H), W2, b2.reshape(1, R),
      route_bias.reshape(1, R))
    return (sel, probs)


# speedup vs baseline: 1.1215x; 1.1215x over previous
import functools

import jax
import jax.numpy as jnp
from jax.experimental import pallas as pl
from jax.experimental.pallas import tpu as pltpu


B, D, H, R = 16384, 2048, 128, 16
TB = 1024
NX = 4
DC = D // NX


def _router_kernel(x0_ref, x1_ref, x2_ref, x3_ref, w1_ref, b1_ref,
                   w2_ref, b2_ref, rb_ref, sel_ref, prob_ref):
    acc = jnp.zeros((TB, H), jnp.float32) + b1_ref[...]
    for k, x_ref in enumerate((x0_ref, x1_ref, x2_ref, x3_ref)):
        acc += jnp.dot(x_ref[...], w1_ref[k * DC:(k + 1) * DC, :],
                       preferred_element_type=jnp.float32)
    h = jnp.maximum(acc, 0.0)
    logits = (jnp.dot(h, w2_ref[...], preferred_element_type=jnp.float32)
              + b2_ref[...] + rb_ref[...])
    m = jnp.max(logits, axis=-1, keepdims=True)
    e = jnp.exp(logits - m)
    prob_ref[...] = e * (1.0 / jnp.sum(e, axis=-1, keepdims=True))
    lane = jax.lax.broadcasted_iota(jnp.int32, logits.shape, 1)
    sel_ref[...] = jnp.min(jnp.where(logits == m, lane, R), axis=-1,
                           keepdims=True)


@functools.partial(jax.jit, static_argnames=())
def kernel(x, W1, b1, W2, b2, route_bias):
    grid = (B // TB,)
    x_specs = [pl.BlockSpec((TB, DC), lambda i, k=k: (i, k))
               for k in range(NX)]
    sel, probs = pl.pallas_call(
        _router_kernel,
        grid=grid,
        in_specs=x_specs + [
            pl.BlockSpec((D, H), lambda i: (0, 0)),
            pl.BlockSpec((1, H), lambda i: (0, 0)),
            pl.BlockSpec((H, R), lambda i: (0, 0)),
            pl.BlockSpec((1, R), lambda i: (0, 0)),
            pl.BlockSpec((1, R), lambda i: (0, 0)),
        ],
        out_specs=[
            pl.BlockSpec((TB, 1), lambda i: (i, 0)),
            pl.BlockSpec((TB, R), lambda i: (i, 0)),
        ],
        out_shape=[
            jax.ShapeDtypeStruct((B, 1), jnp.int32),
            jax.ShapeDtypeStruct((B, R), jnp.float32),
        ],
        compiler_params=pltpu.CompilerParams(
            dimension_semantics=("parallel",)),
    )(x, x, x, x, W1, b1.reshape(1, H), W2, b2.reshape(1, R),
      route_bias.reshape(1, R))
    return (sel.reshape(B), probs)


# speedup vs baseline: 1.1218x; 1.0003x over previous
import functools

import jax
import jax.numpy as jnp
from jax.experimental import pallas as pl
from jax.experimental.pallas import tpu as pltpu


B, D, H, R = 16384, 2048, 128, 16
TB = 1024
NX = 4
RC = TB // NX


def _router_kernel(x0_ref, x1_ref, x2_ref, x3_ref, w1_ref, b1_ref,
                   w2_ref, b2_ref, rb_ref, sel_ref, prob_ref):
    x = jnp.concatenate(
        [x0_ref[...], x1_ref[...], x2_ref[...], x3_ref[...]], axis=0)
    h = jnp.maximum(
        jnp.dot(x, w1_ref[...], preferred_element_type=jnp.float32)
        + b1_ref[...], 0.0)
    logits = (jnp.dot(h, w2_ref[...], preferred_element_type=jnp.float32)
              + b2_ref[...] + rb_ref[...])
    m = jnp.max(logits, axis=-1, keepdims=True)
    e = jnp.exp(logits - m)
    prob_ref[...] = e * (1.0 / jnp.sum(e, axis=-1, keepdims=True))
    lane = jax.lax.broadcasted_iota(jnp.int32, logits.shape, 1)
    sel_ref[...] = jnp.min(jnp.where(logits == m, lane, R), axis=-1,
                           keepdims=True)


@functools.partial(jax.jit, static_argnames=())
def kernel(x, W1, b1, W2, b2, route_bias):
    grid = (B // TB,)
    x_specs = [pl.BlockSpec((RC, D), lambda i, k=k: (NX * i + k, 0))
               for k in range(NX)]
    sel, probs = pl.pallas_call(
        _router_kernel,
        grid=grid,
        in_specs=x_specs + [
            pl.BlockSpec((D, H), lambda i: (0, 0)),
            pl.BlockSpec((1, H), lambda i: (0, 0)),
            pl.BlockSpec((H, R), lambda i: (0, 0)),
            pl.BlockSpec((1, R), lambda i: (0, 0)),
            pl.BlockSpec((1, R), lambda i: (0, 0)),
        ],
        out_specs=[
            pl.BlockSpec((TB, 1), lambda i: (i, 0)),
            pl.BlockSpec((TB, R), lambda i: (i, 0)),
        ],
        out_shape=[
            jax.ShapeDtypeStruct((B, 1), jnp.int32),
            jax.ShapeDtypeStruct((B, R), jnp.float32),
        ],
        compiler_params=pltpu.CompilerParams(
            dimension_semantics=("parallel",)),
    )(x, x, x, x, W1, b1.reshape(1, H), W2, b2.reshape(1, R),
      route_bias.reshape(1, R))
    return (sel.reshape(B), probs)
